# baseline (speedup 1.0000x reference)
import functools

import jax
import jax.numpy as jnp
from jax import lax
from jax.experimental import pallas as pl
from jax.experimental.pallas import tpu as pltpu
from jax.experimental.pallas import tpu_sc as plsc

_B, _S, _H = 1024, 200, 128
_NC, _NS, _L = 1, 8, 16
_NW = _NC * _NS
_BPW = _B // _NW


def _gather_body(table_hbm, pos1_hbm, pos2_hbm, out1_hbm, out2_hbm,
                 idx_v, rows_v, sem_i, sem_g, sem_s):
    wid = lax.axis_index("s") * _NC + lax.axis_index("c")
    base = wid * _BPW

    ld1 = pltpu.async_copy(pos1_hbm.at[pl.ds(base, _BPW)],
                           idx_v.at[pl.ds(0, _BPW)], sem_i)
    ld2 = pltpu.async_copy(pos2_hbm.at[pl.ds(base, _BPW)],
                           idx_v.at[pl.ds(_BPW, _BPW)], sem_i)

    lanes = lax.iota(jnp.int32, _L) * _S

    ld1.wait()
    for j in range(_BPW // _L):
        sl = pl.ds(j * _L, _L)
        idx_v[sl] = idx_v[sl] + ((base + j * _L) * _S + lanes)
    g1 = pltpu.async_copy(table_hbm.at[idx_v.at[pl.ds(0, _BPW)]],
                          rows_v.at[pl.ds(0, _BPW)], sem_g)

    ld2.wait()
    for j in range(_BPW // _L):
        sl = pl.ds(_BPW + j * _L, _L)
        idx_v[sl] = idx_v[sl] + ((base + j * _L) * _S + lanes)
    g2 = pltpu.async_copy(table_hbm.at[idx_v.at[pl.ds(_BPW, _BPW)]],
                          rows_v.at[pl.ds(_BPW, _BPW)], sem_g)

    g1.wait()
    st1 = pltpu.async_copy(rows_v.at[pl.ds(0, _BPW)],
                           out1_hbm.at[pl.ds(base, _BPW)], sem_s)
    g2.wait()
    st2 = pltpu.async_copy(rows_v.at[pl.ds(_BPW, _BPW)],
                           out2_hbm.at[pl.ds(base, _BPW)], sem_s)
    st1.wait()
    st2.wait()


@functools.cache
def _entity_gather():
    return pl.kernel(
        _gather_body,
        mesh=plsc.VectorSubcoreMesh(core_axis_name="c", subcore_axis_name="s",
                                    num_cores=_NC, num_subcores=_NS),
        out_type=(
            jax.ShapeDtypeStruct((_B, _H), jnp.float32),
            jax.ShapeDtypeStruct((_B, _H), jnp.float32),
        ),
        scratch_types=[
            pltpu.VMEM((2 * _BPW,), jnp.int32),
            pltpu.VMEM((2 * _BPW, _H), jnp.float32),
            pltpu.SemaphoreType.DMA,
            pltpu.SemaphoreType.DMA,
            pltpu.SemaphoreType.DMA,
        ],
    )


def kernel(token_embs, pos1, pos2, mask):
    del mask
    table = token_embs.reshape(_B * _S, _H)
    p1 = pos1.reshape(_B).astype(jnp.int32)
    p2 = pos2.reshape(_B).astype(jnp.int32)
    hidden1, hidden2 = _entity_gather()(table, p1, p2)
    return (hidden1, hidden2)


# speedup vs baseline: 1.0478x; 1.0478x over previous
import functools

import jax
import jax.numpy as jnp
from jax import lax
from jax.experimental import pallas as pl
from jax.experimental.pallas import tpu as pltpu
from jax.experimental.pallas import tpu_sc as plsc

_B, _S, _H = 1024, 200, 128
_NC, _NS, _L = 1, 16, 16
_NW = _NC * _NS
_BPW = _B // _NW
_CHUNK = 32


def _gather_body(table_hbm, pos1_hbm, pos2_hbm, out1_hbm, out2_hbm,
                 idx_v, rows_v, sem_i, sem_g, sem_s):
    wid = lax.axis_index("s") * _NC + lax.axis_index("c")
    base = wid * _BPW

    ld1 = pltpu.async_copy(pos1_hbm.at[pl.ds(base, _BPW)],
                           idx_v.at[pl.ds(0, _BPW)], sem_i)
    ld2 = pltpu.async_copy(pos2_hbm.at[pl.ds(base, _BPW)],
                           idx_v.at[pl.ds(_BPW, _BPW)], sem_i)

    lanes = lax.iota(jnp.int32, _L) * _S
    nch = _BPW // _CHUNK
    gathers = []

    ld1.wait()
    for j in range(_BPW // _L):
        sl = pl.ds(j * _L, _L)
        idx_v[sl] = idx_v[sl] + ((base + j * _L) * _S + lanes)
    for c in range(nch):
        off = c * _CHUNK
        gathers.append(pltpu.async_copy(
            table_hbm.at[idx_v.at[pl.ds(off, _CHUNK)]],
            rows_v.at[pl.ds(off, _CHUNK)], sem_g))

    ld2.wait()
    for j in range(_BPW // _L):
        sl = pl.ds(_BPW + j * _L, _L)
        idx_v[sl] = idx_v[sl] + ((base + j * _L) * _S + lanes)
    for c in range(nch):
        off = _BPW + c * _CHUNK
        gathers.append(pltpu.async_copy(
            table_hbm.at[idx_v.at[pl.ds(off, _CHUNK)]],
            rows_v.at[pl.ds(off, _CHUNK)], sem_g))

    stores = []
    for k, g in enumerate(gathers):
        g.wait()
        out_hbm = out1_hbm if k < nch else out2_hbm
        off = (k % nch) * _CHUNK
        stores.append(pltpu.async_copy(
            rows_v.at[pl.ds(k * _CHUNK, _CHUNK)],
            out_hbm.at[pl.ds(base + off, _CHUNK)], sem_s))
    for st in stores:
        st.wait()


@functools.cache
def _entity_gather():
    return pl.kernel(
        _gather_body,
        mesh=plsc.VectorSubcoreMesh(core_axis_name="c", subcore_axis_name="s",
                                    num_cores=_NC, num_subcores=_NS),
        out_type=(
            jax.ShapeDtypeStruct((_B, _H), jnp.float32),
            jax.ShapeDtypeStruct((_B, _H), jnp.float32),
        ),
        scratch_types=[
            pltpu.VMEM((2 * _BPW,), jnp.int32),
            pltpu.VMEM((2 * _BPW, _H), jnp.float32),
            pltpu.SemaphoreType.DMA,
            pltpu.SemaphoreType.DMA,
            pltpu.SemaphoreType.DMA,
        ],
    )


def kernel(token_embs, pos1, pos2, mask):
    del mask
    table = token_embs.reshape(_B * _S, _H)
    p1 = pos1.reshape(_B).astype(jnp.int32)
    p2 = pos2.reshape(_B).astype(jnp.int32)
    hidden1, hidden2 = _entity_gather()(table, p1, p2)
    return (hidden1, hidden2)


# speedup vs baseline: 1.0532x; 1.0051x over previous
import functools

import jax
import jax.numpy as jnp
from jax import lax
from jax.experimental import pallas as pl
from jax.experimental.pallas import tpu as pltpu
from jax.experimental.pallas import tpu_sc as plsc

_B, _S, _H = 1024, 200, 128
_NC, _NS, _L = 1, 16, 16
_NW = _NC * _NS
_BPW = _B // _NW
_CHUNK = 16


def _gather_body(table_hbm, pos1_hbm, pos2_hbm, out1_hbm, out2_hbm,
                 idx_v, rows_v, sem_i, sem_g, sem_s):
    wid = lax.axis_index("s") * _NC + lax.axis_index("c")
    base = wid * _BPW

    ld1 = pltpu.async_copy(pos1_hbm.at[pl.ds(base, _BPW)],
                           idx_v.at[pl.ds(0, _BPW)], sem_i)
    ld2 = pltpu.async_copy(pos2_hbm.at[pl.ds(base, _BPW)],
                           idx_v.at[pl.ds(_BPW, _BPW)], sem_i)

    lanes = lax.iota(jnp.int32, _L) * _S
    nch = _BPW // _CHUNK
    gathers = []

    ld1.wait()
    for j in range(_BPW // _L):
        sl = pl.ds(j * _L, _L)
        idx_v[sl] = idx_v[sl] + ((base + j * _L) * _S + lanes)
    for c in range(nch):
        off = c * _CHUNK
        gathers.append(pltpu.async_copy(
            table_hbm.at[idx_v.at[pl.ds(off, _CHUNK)]],
            rows_v.at[pl.ds(off, _CHUNK)], sem_g))

    ld2.wait()
    for j in range(_BPW // _L):
        sl = pl.ds(_BPW + j * _L, _L)
        idx_v[sl] = idx_v[sl] + ((base + j * _L) * _S + lanes)
    for c in range(nch):
        off = _BPW + c * _CHUNK
        gathers.append(pltpu.async_copy(
            table_hbm.at[idx_v.at[pl.ds(off, _CHUNK)]],
            rows_v.at[pl.ds(off, _CHUNK)], sem_g))

    stores = []
    for k, g in enumerate(gathers):
        g.wait()
        out_hbm = out1_hbm if k < nch else out2_hbm
        off = (k % nch) * _CHUNK
        stores.append(pltpu.async_copy(
            rows_v.at[pl.ds(k * _CHUNK, _CHUNK)],
            out_hbm.at[pl.ds(base + off, _CHUNK)], sem_s))
    for st in stores:
        st.wait()


@functools.cache
def _entity_gather():
    return pl.kernel(
        _gather_body,
        mesh=plsc.VectorSubcoreMesh(core_axis_name="c", subcore_axis_name="s",
                                    num_cores=_NC, num_subcores=_NS),
        out_type=(
            jax.ShapeDtypeStruct((_B, _H), jnp.float32),
            jax.ShapeDtypeStruct((_B, _H), jnp.float32),
        ),
        scratch_types=[
            pltpu.VMEM((2 * _BPW,), jnp.int32),
            pltpu.VMEM((2 * _BPW, _H), jnp.float32),
            pltpu.SemaphoreType.DMA,
            pltpu.SemaphoreType.DMA,
            pltpu.SemaphoreType.DMA,
        ],
    )


def kernel(token_embs, pos1, pos2, mask):
    del mask
    table = token_embs.reshape(_B * _S, _H)
    p1 = pos1.reshape(_B).astype(jnp.int32)
    p2 = pos2.reshape(_B).astype(jnp.int32)
    hidden1, hidden2 = _entity_gather()(table, p1, p2)
    return (hidden1, hidden2)
